# baseline (speedup 1.0000x reference)
import functools

import jax
import jax.numpy as jnp
from jax import lax
from jax.experimental import pallas as pl
from jax.experimental.pallas import tpu as pltpu
from jax.experimental.pallas import tpu_sc as plsc

N = 10000
E = 160000
DF = 128
DE = 16
H0 = 8
IH = 10

NC, NS = 2, 16
NW = NC * NS
EPW = 5120
EPAD = NW * EPW
NAG = 10112
RPT = NAG // NS

_f32 = jnp.float32


def _lane():
    return lax.iota(jnp.int32, 16)


def _zz_segments(col_lo, src_of):
    segs = []
    for l in range(16):
        s = src_of(col_lo + l)
        if not segs or segs[-1][1] != s:
            segs.append((l, s))
    return segs


def _build_zz(segs, zrow, zlane_of_k):
    lane = _lane()

    def val(spec):
        if isinstance(spec, tuple):
            return jnp.full((16,), zrow[zlane_of_k(spec[1])], _f32)
        return jnp.full((16,), spec, _f32)

    zz = val(segs[0][1])
    for b, spec in segs[1:]:
        zz = jnp.where(lane >= b, val(spec), zz)
    return zz


def _make_sc_layer(ch, tw, zw, mw, fold_pairs, nch0, nch1):
    assert NS * (nch0 + nch1) * ch == EPAD
    nj = ch // 128
    mesh = plsc.VectorSubcoreMesh(core_axis_name="c", subcore_axis_name="s")

    @functools.partial(
        pl.kernel,
        out_type=jax.ShapeDtypeStruct((NC, NAG, mw), _f32),
        mesh=mesh,
        compiler_params=pltpu.CompilerParams(use_tc_tiling_on_sc=False),
        scratch_types=[
            pltpu.VMEM((nj, 128), jnp.int32),
            pltpu.VMEM((nj, 128), jnp.int32),
            pltpu.VMEM((ch, zw), _f32),
            pltpu.VMEM((ch, tw), _f32),
            pltpu.VMEM((ch, mw), _f32),
            pltpu.VMEM_SHARED((NAG, mw), _f32),
            pltpu.SemaphoreType.DMA,
        ],
    )
    def sc_layer(t_hbm, zx_hbm, src_hbm, dst_hbm, agg_hbm,
                 src_v, dst_v, zx_v, rows_v, msg_v, agg_sh, sem):
        c = lax.axis_index("c")
        s = lax.axis_index("s")
        cbase = jnp.where(c == 0, s * nch0, NS * nch0 + s * nch1)
        nch_me = jnp.where(c == 0, nch0, nch1)

        @plsc.parallel_loop(0, ch, 1, unroll=8)
        def zrow_body(r):
            for j in range(mw // 16):
                msg_v[r, pl.ds(j * 16, 16)] = jnp.zeros((16,), _f32)

        done = 0
        while done < RPT:
            n = min(ch, RPT - done)
            pltpu.sync_copy(msg_v.at[pl.ds(0, n)],
                            agg_sh.at[pl.ds(s * RPT + done, n)])
            done += n
        plsc.subcore_barrier()

        def chunk(ci, carry):
            ebase = (cbase + ci) * ch
            rbase = (cbase + ci) * nj
            pltpu.sync_copy(src_hbm.at[pl.ds(rbase, nj)], src_v)
            pltpu.sync_copy(dst_hbm.at[pl.ds(rbase, nj)], dst_v)
            pltpu.sync_copy(zx_hbm.at[pl.ds(ebase, ch)], zx_v)
            cps = [pltpu.async_copy(t_hbm.at[src_v.at[j]],
                                    rows_v.at[pl.ds(j * 128, 128)], sem)
                   for j in range(nj)]
            for cp in cps:
                cp.wait()

            @plsc.parallel_loop(0, ch, 1, unroll=4)
            def edge(e):
                if not fold_pairs:
                    acc = rows_v[e, pl.ds(0, 16)] * zx_v[e, pl.ds(0, 16)]
                    for j in range(1, 6):
                        acc = acc + rows_v[e, pl.ds(j * 16, 16)] *                             zx_v[e, pl.ds(j * 16, 16)]
                    msg_v[e, pl.ds(0, 16)] = acc
                else:
                    prods = [rows_v[e, pl.ds(j * 16, 16)] *
                             zx_v[e, pl.ds(j * 16, 16)]
                             for j in range(7)]
                    for j in range(5):
                        v = prods[j] + prods[j + 5] if j < 2 else prods[j]
                        msg_v[e, pl.ds(j * 16, 16)] = v

            for j in range(nj):
                pltpu.sync_copy(msg_v.at[pl.ds(j * 128, 128)],
                                agg_sh.at[dst_v.at[j]], add=True)
            return carry

        lax.fori_loop(0, nch_me, chunk, 0)
        plsc.subcore_barrier()
        pltpu.sync_copy(agg_sh.at[pl.ds(s * RPT, RPT)],
                        agg_hbm.at[c].at[pl.ds(s * RPT, RPT)])

    return sc_layer


def _make_sc_ep(ch, nch0, nch1):
    assert NS * (nch0 + nch1) * ch == EPAD
    nj = ch // 128
    mesh = plsc.VectorSubcoreMesh(core_axis_name="c", subcore_axis_name="s")

    @functools.partial(
        pl.kernel,
        out_type=jax.ShapeDtypeStruct((EPAD, 16), _f32),
        mesh=mesh,
        compiler_params=pltpu.CompilerParams(use_tc_tiling_on_sc=False),
        scratch_types=[
            pltpu.VMEM((nj, 128), jnp.int32),
            pltpu.VMEM((ch, 16), _f32),
            pltpu.VMEM((ch, 16), _f32),
            pltpu.VMEM((ch, 16), _f32),
            pltpu.SemaphoreType.DMA,
        ],
    )
    def sc_ep(q_hbm, a_hbm, src_hbm, out_hbm, src_v, a_v, q_v, r_v, sem):
        c = lax.axis_index("c")
        s = lax.axis_index("s")
        cbase = jnp.where(c == 0, s * nch0, NS * nch0 + s * nch1)
        nch_me = jnp.where(c == 0, nch0, nch1)

        def chunk(ci, carry):
            ebase = (cbase + ci) * ch
            rbase = (cbase + ci) * nj
            pltpu.sync_copy(src_hbm.at[pl.ds(rbase, nj)], src_v)
            pltpu.sync_copy(a_hbm.at[pl.ds(ebase, ch)], a_v)
            cps = [pltpu.async_copy(q_hbm.at[src_v.at[j]],
                                    q_v.at[pl.ds(j * 128, 128)], sem)
                   for j in range(nj)]
            for cp in cps:
                cp.wait()

            @plsc.parallel_loop(0, ch, 1, unroll=8)
            def edge(e):
                v = a_v[e, pl.ds(0, 16)] + q_v[e, pl.ds(0, 16)]
                r_v[e, pl.ds(0, 16)] = jnp.maximum(v, 0.0)

            pltpu.sync_copy(r_v, out_hbm.at[pl.ds(ebase, ch)])
            return carry

        lax.fori_loop(0, nch_me, chunk, 0)

    return sc_ep


_sc_l0 = _make_sc_layer(ch=256, tw=128, zw=128, mw=16, fold_pairs=False,
                        nch0=20, nch1=20)
_sc_l1 = _make_sc_layer(ch=128, tw=112, zw=128, mw=80, fold_pairs=True,
                        nch0=40, nch1=40)
_sc_ep = _make_sc_ep(ch=512, nch0=10, nch1=10)


def _zx_k(z_ref, s_ref, zx_ref):
    be = z_ref.shape[0]
    zh = jnp.concatenate([z_ref[...], jnp.ones((be, 1), _f32),
                          jnp.zeros((be, 5), _f32)], axis=1)
    zx_ref[...] = jnp.dot(zh, s_ref[...], preferred_element_type=_f32,
                          precision=lax.Precision.HIGHEST)


def _zx_expand(z, s, zw):
    be, grid = 1600, E // 1600
    return pl.pallas_call(
        _zx_k,
        grid=(grid,),
        in_specs=[
            pl.BlockSpec((be, 10), lambda i: (i, 0)),
            pl.BlockSpec((16, s.shape[1]), lambda i: (0, 0)),
        ],
        out_specs=pl.BlockSpec((be, zw), lambda i: (i, 0)),
        out_shape=jax.ShapeDtypeStruct((EPAD, zw), _f32),
    )(z, s)


def kernel(x, edge_attr, edge_index, nn0_W1, nn0_b1, nn0_W2, nn0_b2, root0,
           bias0, nn1_W1, nn1_b1, nn1_W2, nn1_b2, root1, bias1, ep_W1, ep_b1,
           ep_W2, ep_b2):
    src = jnp.concatenate([edge_index[0], jnp.zeros((EPAD - E,), jnp.int32)])
    dst = jnp.concatenate([edge_index[1],
                           jnp.full((EPAD - E,), N, jnp.int32)])
    src2d = src.reshape(EPAD // 128, 128)
    dst2d = dst.reshape(EPAD // 128, 128)

    z0 = jnp.maximum(edge_attr @ nn0_W1 + nn0_b1, 0.0)
    z1 = jnp.maximum(edge_attr @ nn1_W1 + nn1_b1, 0.0)
    epad = EPAD - E

    k0 = jnp.arange(128) // 8
    s0 = (jnp.arange(16)[:, None] == jnp.where(k0 < 10, k0, 10)[None, :]
          ).astype(_f32) * (jnp.arange(128) < 88)[None, :]
    k1 = jnp.arange(128) // 10
    s1 = (jnp.arange(16)[:, None] == jnp.where(k1 < 10, k1, 10)[None, :]
          ).astype(_f32) * (jnp.arange(128) < 110)[None, :]
    zx0 = _zx_expand(z0, s0, 128)
    zx1 = _zx_expand(z1, s1, 128)

    m0 = nn0_W2.reshape(IH, DF, H0).transpose(1, 0, 2).reshape(DF, IH * H0)
    t0 = x @ jnp.concatenate([m0, nn0_b2.reshape(DF, H0),
                              jnp.zeros((DF, 40), _f32)], axis=1)

    agg0 = _sc_l0(t0, zx0, src2d, dst2d)
    a0 = agg0[0] + agg0[1]
    h1 = jnp.maximum(a0[:N, :8] + a0[:N, 8:] + x @ root0 + bias0, 0.0)

    m1 = nn1_W2.reshape(IH, H0, IH).transpose(1, 0, 2).reshape(H0, IH * IH)
    t1 = jnp.concatenate(
        [h1 @ jnp.concatenate([m1, nn1_b2.reshape(H0, IH)], axis=1),
         jnp.zeros((N, 2), _f32)], axis=1)

    agg1 = _sc_l1(t1, zx1, src2d, dst2d)
    a1 = agg1[0][:N] + agg1[1][:N]
    pre = h1 @ root1 + bias1
    for k in range(8):
        pre = pre + a1[:, k * 10:k * 10 + 10]
    h2 = jnp.maximum(pre, 0.0)

    a_e = jnp.concatenate([edge_attr @ ep_W1[:DE] + ep_b1,
                           jnp.zeros((E, 6), _f32)], axis=1)
    a_e = jnp.pad(a_e, ((0, epad), (0, 0)))
    q = jnp.concatenate([h2 @ ep_W1[DE:], jnp.zeros((N, 6), _f32)], axis=1)

    s2d = _sc_ep(q, a_e, src2d)
    scores = s2d[:E, :10] @ ep_W2 + ep_b2
    return scores[:, 0]


# speedup vs baseline: 1.0181x; 1.0181x over previous
import functools

import jax
import jax.numpy as jnp
from jax import lax
from jax.experimental import pallas as pl
from jax.experimental.pallas import tpu as pltpu
from jax.experimental.pallas import tpu_sc as plsc

N = 10000
E = 160000
DF = 128
DE = 16
H0 = 8
IH = 10

NC, NS = 2, 16
NW = NC * NS
EPW = 5120
EPAD = NW * EPW
NAG = 10112
RPT = NAG // NS

_f32 = jnp.float32


def _lane():
    return lax.iota(jnp.int32, 16)


def _zz_segments(col_lo, src_of):
    segs = []
    for l in range(16):
        s = src_of(col_lo + l)
        if not segs or segs[-1][1] != s:
            segs.append((l, s))
    return segs


def _build_zz(segs, zrow, zlane_of_k):
    lane = _lane()

    def val(spec):
        if isinstance(spec, tuple):
            return jnp.full((16,), zrow[zlane_of_k(spec[1])], _f32)
        return jnp.full((16,), spec, _f32)

    zz = val(segs[0][1])
    for b, spec in segs[1:]:
        zz = jnp.where(lane >= b, val(spec), zz)
    return zz


def _make_sc_layer(ch, tw, zw, mw, fold_pairs, nch0, nch1):
    assert NS * (nch0 + nch1) * ch == EPAD
    nj = ch // 128
    mesh = plsc.VectorSubcoreMesh(core_axis_name="c", subcore_axis_name="s")

    @functools.partial(
        pl.kernel,
        out_type=jax.ShapeDtypeStruct((NC, NAG, mw), _f32),
        mesh=mesh,
        compiler_params=pltpu.CompilerParams(use_tc_tiling_on_sc=False),
        scratch_types=[
            pltpu.VMEM((nj, 128), jnp.int32),
            pltpu.VMEM((nj, 128), jnp.int32),
            pltpu.VMEM((ch, zw), _f32),
            pltpu.VMEM((ch, tw), _f32),
            pltpu.VMEM((ch, mw), _f32),
            pltpu.VMEM_SHARED((NAG, mw), _f32),
            pltpu.SemaphoreType.DMA,
        ],
    )
    def sc_layer(t_hbm, zx_hbm, src_hbm, dst_hbm, agg_hbm,
                 src_v, dst_v, zx_v, rows_v, msg_v, agg_sh, sem):
        c = lax.axis_index("c")
        s = lax.axis_index("s")
        cbase = jnp.where(c == 0, s * nch0, NS * nch0 + s * nch1)
        nch_me = jnp.where(c == 0, nch0, nch1)

        @plsc.parallel_loop(0, ch, 1, unroll=8)
        def zrow_body(r):
            for j in range(mw // 16):
                msg_v[r, pl.ds(j * 16, 16)] = jnp.zeros((16,), _f32)

        done = 0
        while done < RPT:
            n = min(ch, RPT - done)
            pltpu.sync_copy(msg_v.at[pl.ds(0, n)],
                            agg_sh.at[pl.ds(s * RPT + done, n)])
            done += n
        plsc.subcore_barrier()

        def chunk(ci, carry):
            ebase = (cbase + ci) * ch
            rbase = (cbase + ci) * nj
            pltpu.sync_copy(src_hbm.at[pl.ds(rbase, nj)], src_v)
            pltpu.sync_copy(dst_hbm.at[pl.ds(rbase, nj)], dst_v)
            pltpu.sync_copy(zx_hbm.at[pl.ds(ebase, ch)], zx_v)
            cps = [pltpu.async_copy(t_hbm.at[src_v.at[j]],
                                    rows_v.at[pl.ds(j * 128, 128)], sem)
                   for j in range(nj)]
            for cp in cps:
                cp.wait()

            @plsc.parallel_loop(0, ch, 1, unroll=4)
            def edge(e):
                if not fold_pairs:
                    acc = rows_v[e, pl.ds(0, 16)] * zx_v[e, pl.ds(0, 16)]
                    for j in range(1, 6):
                        acc = acc + rows_v[e, pl.ds(j * 16, 16)] *                             zx_v[e, pl.ds(j * 16, 16)]
                    msg_v[e, pl.ds(0, 16)] = acc
                else:
                    prods = [rows_v[e, pl.ds(j * 16, 16)] *
                             zx_v[e, pl.ds(j * 16, 16)]
                             for j in range(tw // 16)]
                    for j in range(5):
                        v = prods[j] + prods[j + 5] if j < 2 else prods[j]
                        msg_v[e, pl.ds(j * 16, 16)] = v

            for j in range(nj):
                pltpu.sync_copy(msg_v.at[pl.ds(j * 128, 128)],
                                agg_sh.at[dst_v.at[j]], add=True)
            return carry

        lax.fori_loop(0, nch_me, chunk, 0)
        plsc.subcore_barrier()
        pltpu.sync_copy(agg_sh.at[pl.ds(s * RPT, RPT)],
                        agg_hbm.at[c].at[pl.ds(s * RPT, RPT)])

    return sc_layer


def _make_sc_ep(ch, nch0, nch1):
    assert NS * (nch0 + nch1) * ch == EPAD
    nj = ch // 128
    mesh = plsc.VectorSubcoreMesh(core_axis_name="c", subcore_axis_name="s")

    @functools.partial(
        pl.kernel,
        out_type=jax.ShapeDtypeStruct((EPAD, 16), _f32),
        mesh=mesh,
        compiler_params=pltpu.CompilerParams(use_tc_tiling_on_sc=False),
        scratch_types=[
            pltpu.VMEM((nj, 128), jnp.int32),
            pltpu.VMEM((ch, 16), _f32),
            pltpu.VMEM((ch, 16), _f32),
            pltpu.VMEM((ch, 16), _f32),
            pltpu.SemaphoreType.DMA,
        ],
    )
    def sc_ep(q_hbm, a_hbm, src_hbm, out_hbm, src_v, a_v, q_v, r_v, sem):
        c = lax.axis_index("c")
        s = lax.axis_index("s")
        cbase = jnp.where(c == 0, s * nch0, NS * nch0 + s * nch1)
        nch_me = jnp.where(c == 0, nch0, nch1)

        def chunk(ci, carry):
            ebase = (cbase + ci) * ch
            rbase = (cbase + ci) * nj
            pltpu.sync_copy(src_hbm.at[pl.ds(rbase, nj)], src_v)
            pltpu.sync_copy(a_hbm.at[pl.ds(ebase, ch)], a_v)
            cps = [pltpu.async_copy(q_hbm.at[src_v.at[j]],
                                    q_v.at[pl.ds(j * 128, 128)], sem)
                   for j in range(nj)]
            for cp in cps:
                cp.wait()

            @plsc.parallel_loop(0, ch, 1, unroll=8)
            def edge(e):
                v = a_v[e, pl.ds(0, 16)] + q_v[e, pl.ds(0, 16)]
                r_v[e, pl.ds(0, 16)] = jnp.maximum(v, 0.0)

            pltpu.sync_copy(r_v, out_hbm.at[pl.ds(ebase, ch)])
            return carry

        lax.fori_loop(0, nch_me, chunk, 0)

    return sc_ep


_sc_l0 = _make_sc_layer(ch=256, tw=128, zw=128, mw=16, fold_pairs=False,
                        nch0=20, nch1=20)
_sc_l1 = _make_sc_layer(ch=256, tw=112, zw=112, mw=80, fold_pairs=True,
                        nch0=20, nch1=20)
_sc_ep = _make_sc_ep(ch=512, nch0=10, nch1=10)


def _zx_k(z_ref, s_ref, zx_ref):
    be = z_ref.shape[0]
    zh = jnp.concatenate([z_ref[...], jnp.ones((be, 1), _f32),
                          jnp.zeros((be, 5), _f32)], axis=1)
    zx_ref[...] = jnp.dot(zh, s_ref[...], preferred_element_type=_f32,
                          precision=lax.Precision.HIGHEST)


def _zx_expand(z, s, zw):
    be, grid = 1600, E // 1600
    return pl.pallas_call(
        _zx_k,
        grid=(grid,),
        in_specs=[
            pl.BlockSpec((be, 10), lambda i: (i, 0)),
            pl.BlockSpec((16, s.shape[1]), lambda i: (0, 0)),
        ],
        out_specs=pl.BlockSpec((be, zw), lambda i: (i, 0)),
        out_shape=jax.ShapeDtypeStruct((EPAD, zw), _f32),
    )(z, s)


def kernel(x, edge_attr, edge_index, nn0_W1, nn0_b1, nn0_W2, nn0_b2, root0,
           bias0, nn1_W1, nn1_b1, nn1_W2, nn1_b2, root1, bias1, ep_W1, ep_b1,
           ep_W2, ep_b2):
    src = jnp.concatenate([edge_index[0], jnp.zeros((EPAD - E,), jnp.int32)])
    dst = jnp.concatenate([edge_index[1],
                           jnp.full((EPAD - E,), N, jnp.int32)])
    src2d = src.reshape(EPAD // 128, 128)
    dst2d = dst.reshape(EPAD // 128, 128)

    z0 = jnp.maximum(edge_attr @ nn0_W1 + nn0_b1, 0.0)
    z1 = jnp.maximum(edge_attr @ nn1_W1 + nn1_b1, 0.0)
    epad = EPAD - E

    k0 = jnp.arange(128) // 8
    s0 = (jnp.arange(16)[:, None] == jnp.where(k0 < 10, k0, 10)[None, :]
          ).astype(_f32) * (jnp.arange(128) < 88)[None, :]
    k1 = jnp.arange(112) // 10
    s1 = (jnp.arange(16)[:, None] == jnp.where(k1 < 10, k1, 10)[None, :]
          ).astype(_f32) * (jnp.arange(112) < 110)[None, :]
    zx0 = _zx_expand(z0, s0, 128)
    zx1 = _zx_expand(z1, s1, 112)

    m0 = nn0_W2.reshape(IH, DF, H0).transpose(1, 0, 2).reshape(DF, IH * H0)
    t0 = x @ jnp.concatenate([m0, nn0_b2.reshape(DF, H0),
                              jnp.zeros((DF, 40), _f32)], axis=1)

    agg0 = _sc_l0(t0, zx0, src2d, dst2d)
    a0 = agg0[0] + agg0[1]
    h1 = jnp.maximum(a0[:N, :8] + a0[:N, 8:] + x @ root0 + bias0, 0.0)

    m1 = nn1_W2.reshape(IH, H0, IH).transpose(1, 0, 2).reshape(H0, IH * IH)
    t1 = jnp.concatenate(
        [h1 @ jnp.concatenate([m1, nn1_b2.reshape(H0, IH)], axis=1),
         jnp.zeros((N, 2), _f32)], axis=1)

    agg1 = _sc_l1(t1, zx1, src2d, dst2d)
    a1 = agg1[0][:N] + agg1[1][:N]
    pre = h1 @ root1 + bias1
    for k in range(8):
        pre = pre + a1[:, k * 10:k * 10 + 10]
    h2 = jnp.maximum(pre, 0.0)

    a_e = jnp.concatenate([edge_attr @ ep_W1[:DE] + ep_b1,
                           jnp.zeros((E, 6), _f32)], axis=1)
    a_e = jnp.pad(a_e, ((0, epad), (0, 0)))
    q = jnp.concatenate([h2 @ ep_W1[DE:], jnp.zeros((N, 6), _f32)], axis=1)

    s2d = _sc_ep(q, a_e, src2d)
    scores = s2d[:E, :10] @ ep_W2 + ep_b2
    return scores[:, 0]
